# Initial kernel scaffold; baseline (speedup 1.0000x reference)
import functools

import jax
import jax.numpy as jnp
from jax import lax
from jax.experimental import pallas as pl
from jax.experimental.pallas import tpu as pltpu
from jax.experimental.pallas import tpu_sc as plsc

_N = 10000
_E = 320000
_D = 128
_G = 128

_NC, _NS = 2, 16
_NW = _NC * _NS
_RPT = 632
_N_PAD = _RPT * _NS
_EW = _E // _NW
_K = 80
_NCHUNK = _EW // _K

_BLK = _RPT
_NBLK = _N_PAD // _BLK

_mesh = plsc.VectorSubcoreMesh(core_axis_name="c", subcore_axis_name="s")


@functools.partial(
    pl.kernel,
    out_type=jax.ShapeDtypeStruct((_NW, _N_PAD), jnp.float32),
    mesh=_mesh,
    scratch_types=[
        pltpu.VMEM((_EW,), jnp.int32),
        pltpu.VMEM((_N_PAD,), jnp.float32),
    ],
)
def _deg_kernel(dst_hbm, out_hbm, dst_v, deg_v):
    c = lax.axis_index("c")
    s = lax.axis_index("s")
    w = c * _NS + s

    def zero_body(i, carry):
        deg_v[pl.ds(i * 16, 16)] = jnp.zeros((16,), jnp.float32)
        return carry

    lax.fori_loop(0, _N_PAD // 16, zero_body, 0)

    pltpu.sync_copy(dst_hbm.at[pl.ds(w * _EW, _EW)], dst_v)
    ones = jnp.full((16,), 1.0, jnp.float32)

    def body(i, carry):
        idx = dst_v[pl.ds(i * 16, 16)]
        plsc.addupdate_scatter(deg_v, [idx], ones)
        return carry

    lax.fori_loop(0, _EW // 16, body, 0)
    pltpu.sync_copy(deg_v, out_hbm.at[w])


@functools.partial(
    pl.kernel,
    out_type=jax.ShapeDtypeStruct((_NC, _N_PAD, _D), jnp.float32),
    mesh=_mesh,
    scratch_types=[
        pltpu.VMEM((_K,), jnp.int32),
        pltpu.VMEM((_K,), jnp.int32),
        pltpu.VMEM((_K, _D), jnp.float32),
        pltpu.VMEM_SHARED((_N_PAD, _D), jnp.float32),
        pltpu.SemaphoreType.DMA,
    ],
)
def _agg_kernel(u_hbm, src_hbm, dst_hbm, zeros_hbm, out_hbm,
                src_v, dst_v, rows_v, acc_sh, sem):
    c = lax.axis_index("c")
    s = lax.axis_index("s")
    w = c * _NS + s

    pltpu.sync_copy(zeros_hbm, acc_sh.at[pl.ds(s * _RPT, _RPT)])
    plsc.subcore_barrier()

    ebase = w * _EW

    def body(i, carry):
        base = ebase + i * _K
        pltpu.sync_copy(src_hbm.at[pl.ds(base, _K)], src_v)
        pltpu.async_copy(u_hbm.at[src_v], rows_v, sem).wait()
        pltpu.sync_copy(dst_hbm.at[pl.ds(base, _K)], dst_v)
        pltpu.sync_copy(rows_v, acc_sh.at[dst_v], add=True)
        return carry

    lax.fori_loop(0, _NCHUNK, body, 0)
    plsc.subcore_barrier()
    pltpu.sync_copy(acc_sh.at[pl.ds(s * _RPT, _RPT)],
                    out_hbm.at[c, pl.ds(s * _RPT, _RPT)])


def _dinv_of(deg_ref):
    deg = jnp.sum(deg_ref[...], axis=0) + 1.0
    return lax.rsqrt(deg)


def _dot(a, b):
    return jnp.dot(a, b, preferred_element_type=jnp.float32,
                   precision=lax.Precision.HIGHEST)


def _tca_body(x_ref, deg_ref, w_ref, u_ref):
    dinv = _dinv_of(deg_ref)
    u_ref[...] = _dot(x_ref[...], w_ref[...]) * dinv[:, None]


def _tcb_body(p_ref, u_ref, deg_ref, b_ref, w_ref, h_ref, un_ref):
    dinv = _dinv_of(deg_ref)
    agg = p_ref[0] + p_ref[1] + u_ref[...]
    h = jnp.maximum(agg * dinv[:, None] + b_ref[...], 0.0)
    h_ref[...] = h
    un_ref[...] = _dot(h, w_ref[...]) * dinv[:, None]


def _tcc_body(p_ref, u_ref, deg_ref, b3_ref, h1_ref, h2_ref, wjk_ref,
              bjk_ref, batch_ref, wm1_ref, bm1_ref, wm2_ref, bm2_ref,
              out_ref, g_acc):
    i = pl.program_id(0)

    @pl.when(i == 0)
    def _():
        g_acc[...] = jnp.zeros_like(g_acc)

    dinv = _dinv_of(deg_ref)
    agg = p_ref[0] + p_ref[1] + u_ref[...]
    h3 = jnp.maximum(agg * dinv[:, None] + b3_ref[...], 0.0)
    hjk = (_dot(h1_ref[...], wjk_ref[0]) + _dot(h2_ref[...], wjk_ref[1])
           + _dot(h3, wjk_ref[2]) + bjk_ref[...])
    batch = batch_ref[0, 0, :]
    gids = lax.broadcasted_iota(jnp.int32, (_BLK, _G), 1)
    oh = (batch[:, None] == gids).astype(jnp.float32)
    g_acc[...] += lax.dot_general(oh, hjk, (((0,), (0,)), ((), ())),
                                  preferred_element_type=jnp.float32,
                                  precision=lax.Precision.HIGHEST)

    @pl.when(i == _NBLK - 1)
    def _():
        g = g_acc[...]
        m = jnp.maximum(_dot(g, wm1_ref[...]) + bm1_ref[...], 0.0)
        out_ref[...] = _dot(m, wm2_ref[...]) + bm2_ref[...]


def _row_spec(d=_D):
    return pl.BlockSpec((_BLK, d), lambda i: (i, 0))


def _full(shape):
    return pl.BlockSpec(shape, lambda i: tuple(0 for _ in shape))


_deg_spec = pl.BlockSpec((_NW, _BLK), lambda i: (0, i))
_p_spec = pl.BlockSpec((_NC, _BLK, _D), lambda i: (0, i, 0))

_tca = pl.pallas_call(
    _tca_body,
    grid=(_NBLK,),
    in_specs=[_row_spec(), _deg_spec, _full((_D, _D))],
    out_specs=_row_spec(),
    out_shape=jax.ShapeDtypeStruct((_N_PAD, _D), jnp.float32),
    compiler_params=pltpu.CompilerParams(
        dimension_semantics=("parallel",)),
)

_tcb = pl.pallas_call(
    _tcb_body,
    grid=(_NBLK,),
    in_specs=[_p_spec, _row_spec(), _deg_spec, _full((1, _D)),
              _full((_D, _D))],
    out_specs=[_row_spec(), _row_spec()],
    out_shape=[jax.ShapeDtypeStruct((_N_PAD, _D), jnp.float32),
               jax.ShapeDtypeStruct((_N_PAD, _D), jnp.float32)],
    compiler_params=pltpu.CompilerParams(
        dimension_semantics=("parallel",)),
)

_tcc = pl.pallas_call(
    _tcc_body,
    grid=(_NBLK,),
    in_specs=[_p_spec, _row_spec(), _deg_spec, _full((1, _D)),
              _row_spec(), _row_spec(), _full((3, _D, _D)), _full((1, _D)),
              pl.BlockSpec((1, 1, _BLK), lambda i: (i, 0, 0)),
              _full((_D, _D)), _full((1, _D)), _full((_D, 64)),
              _full((1, 64))],
    out_specs=_full((_G, 64)),
    out_shape=jax.ShapeDtypeStruct((_G, 64), jnp.float32),
    scratch_shapes=[pltpu.VMEM((_G, _D), jnp.float32)],
    compiler_params=pltpu.CompilerParams(
        dimension_semantics=("arbitrary",)),
)


def kernel(x, edge_index, batch, W1, b1, W2, b2, W3, b3, Wjk, bjk,
           Wm1, bm1, Wm2, bm2):
    src = edge_index[0]
    dst = edge_index[1]

    x_pad = jnp.pad(x, ((0, _N_PAD - _N), (0, 0)))
    batch_pad = jnp.pad(batch, (0, _N_PAD - _N),
                        constant_values=jnp.int32(2 ** 30))
    batch3 = batch_pad.reshape(_NBLK, 1, _BLK)
    zeros_tile = jnp.zeros((_RPT, _D), jnp.float32)

    deg_parts = _deg_kernel(dst)
    u1 = _tca(x_pad, deg_parts, W1)
    p1 = _agg_kernel(u1, src, dst, zeros_tile)
    h1, u2 = _tcb(p1, u1, deg_parts, b1.reshape(1, _D), W2)
    p2 = _agg_kernel(u2, src, dst, zeros_tile)
    h2, u3 = _tcb(p2, u2, deg_parts, b2.reshape(1, _D), W3)
    p3 = _agg_kernel(u3, src, dst, zeros_tile)
    out = _tcc(p3, u3, deg_parts, b3.reshape(1, _D), h1, h2,
               Wjk.reshape(3, _D, _D), bjk.reshape(1, _D), batch3,
               Wm1, bm1.reshape(1, _D), Wm2, bm2.reshape(1, 64))
    return out


# speedup vs baseline: 10.2223x; 10.2223x over previous
import functools

import jax
import jax.numpy as jnp
from jax import lax
from jax.experimental import pallas as pl
from jax.experimental.pallas import tpu as pltpu
from jax.experimental.pallas import tpu_sc as plsc

_N = 10000
_E = 320000
_D = 128
_G = 128

_NC, _NS = 2, 16
_NW = _NC * _NS
_RPT = 632
_N_PAD = _RPT * _NS
_EW = _E // _NW
_K = 80
_NCHUNK = _EW // _K

_BLK = _RPT
_NBLK = _N_PAD // _BLK

_mesh = plsc.VectorSubcoreMesh(core_axis_name="c", subcore_axis_name="s")


_DW = _D


@functools.partial(
    pl.kernel,
    out_type=jax.ShapeDtypeStruct((_NC, _N_PAD, _DW), jnp.float32),
    mesh=_mesh,
    scratch_types=[
        pltpu.VMEM((_K,), jnp.int32),
        pltpu.VMEM((_K, _DW), jnp.float32),
        pltpu.VMEM_SHARED((_N_PAD, _DW), jnp.float32),
    ],
)
def _deg_kernel(dst_hbm, ones_hbm, zeros_hbm, out_hbm, dst_v, ones_v, acc_sh):
    c = lax.axis_index("c")
    s = lax.axis_index("s")
    w = c * _NS + s

    pltpu.sync_copy(zeros_hbm, acc_sh.at[pl.ds(s * _RPT, _RPT)])
    pltpu.sync_copy(ones_hbm, ones_v)
    plsc.subcore_barrier()

    ebase = w * _EW

    def body(i, carry):
        pltpu.sync_copy(dst_hbm.at[pl.ds(ebase + i * _K, _K)], dst_v)
        pltpu.sync_copy(ones_v, acc_sh.at[dst_v], add=True)
        return carry

    lax.fori_loop(0, _NCHUNK, body, 0)
    plsc.subcore_barrier()
    pltpu.sync_copy(acc_sh.at[pl.ds(s * _RPT, _RPT)],
                    out_hbm.at[c, pl.ds(s * _RPT, _RPT)])


@functools.partial(
    pl.kernel,
    out_type=jax.ShapeDtypeStruct((_NC, _N_PAD, _D), jnp.float32),
    mesh=_mesh,
    scratch_types=[
        pltpu.VMEM((_K,), jnp.int32),
        pltpu.VMEM((_K,), jnp.int32),
        pltpu.VMEM((_K, _D), jnp.float32),
        pltpu.VMEM_SHARED((_N_PAD, _D), jnp.float32),
        pltpu.SemaphoreType.DMA,
    ],
)
def _agg_kernel(u_hbm, src_hbm, dst_hbm, zeros_hbm, out_hbm,
                src_v, dst_v, rows_v, acc_sh, sem):
    c = lax.axis_index("c")
    s = lax.axis_index("s")
    w = c * _NS + s

    pltpu.sync_copy(zeros_hbm, acc_sh.at[pl.ds(s * _RPT, _RPT)])
    plsc.subcore_barrier()

    ebase = w * _EW

    def body(i, carry):
        base = ebase + i * _K
        pltpu.sync_copy(src_hbm.at[pl.ds(base, _K)], src_v)
        pltpu.async_copy(u_hbm.at[src_v], rows_v, sem).wait()
        pltpu.sync_copy(dst_hbm.at[pl.ds(base, _K)], dst_v)
        pltpu.sync_copy(rows_v, acc_sh.at[dst_v], add=True)
        return carry

    lax.fori_loop(0, _NCHUNK, body, 0)
    plsc.subcore_barrier()
    pltpu.sync_copy(acc_sh.at[pl.ds(s * _RPT, _RPT)],
                    out_hbm.at[c, pl.ds(s * _RPT, _RPT)])


def _dinv_body(deg_ref, dinv_ref):
    deg = deg_ref[0, :, 0] + deg_ref[1, :, 0] + 1.0
    dinv_ref[...] = lax.rsqrt(deg)[:, None]


_dinv_kernel = pl.pallas_call(
    _dinv_body,
    out_shape=jax.ShapeDtypeStruct((_N_PAD, 1), jnp.float32),
)


def _dot(a, b):
    return jnp.dot(a, b, preferred_element_type=jnp.float32,
                   precision=lax.Precision.HIGHEST)


def _tca_body(x_ref, dinv_ref, w_ref, u_ref):
    u_ref[...] = _dot(x_ref[...], w_ref[...]) * dinv_ref[...]


def _tcb_body(p_ref, u_ref, dinv_ref, b_ref, w_ref, h_ref, un_ref):
    dinv = dinv_ref[...]
    agg = p_ref[0] + p_ref[1] + u_ref[...]
    h = jnp.maximum(agg * dinv + b_ref[...], 0.0)
    h_ref[...] = h
    un_ref[...] = _dot(h, w_ref[...]) * dinv


def _tcc_body(p_ref, u_ref, dinv_ref, b3_ref, h1_ref, h2_ref, wjk_ref,
              bjk_ref, batch_ref, wm1_ref, bm1_ref, wm2_ref, bm2_ref,
              out_ref, g_acc):
    i = pl.program_id(0)

    @pl.when(i == 0)
    def _():
        g_acc[...] = jnp.zeros_like(g_acc)

    agg = p_ref[0] + p_ref[1] + u_ref[...]
    h3 = jnp.maximum(agg * dinv_ref[...] + b3_ref[...], 0.0)
    hjk = (_dot(h1_ref[...], wjk_ref[0]) + _dot(h2_ref[...], wjk_ref[1])
           + _dot(h3, wjk_ref[2]) + bjk_ref[...])
    batch = batch_ref[0, 0, :]
    gids = lax.broadcasted_iota(jnp.int32, (_BLK, _G), 1)
    oh = (batch[:, None] == gids).astype(jnp.float32)
    g_acc[...] += lax.dot_general(oh, hjk, (((0,), (0,)), ((), ())),
                                  preferred_element_type=jnp.float32,
                                  precision=lax.Precision.HIGHEST)

    @pl.when(i == _NBLK - 1)
    def _():
        g = g_acc[...]
        m = jnp.maximum(_dot(g, wm1_ref[...]) + bm1_ref[...], 0.0)
        out_ref[...] = _dot(m, wm2_ref[...]) + bm2_ref[...]


def _row_spec(d=_D):
    return pl.BlockSpec((_BLK, d), lambda i: (i, 0))


def _full(shape):
    return pl.BlockSpec(shape, lambda i: tuple(0 for _ in shape))


_dinv_spec = pl.BlockSpec((_BLK, 1), lambda i: (i, 0))
_p_spec = pl.BlockSpec((_NC, _BLK, _D), lambda i: (0, i, 0))

_tca = pl.pallas_call(
    _tca_body,
    grid=(_NBLK,),
    in_specs=[_row_spec(), _dinv_spec, _full((_D, _D))],
    out_specs=_row_spec(),
    out_shape=jax.ShapeDtypeStruct((_N_PAD, _D), jnp.float32),
    compiler_params=pltpu.CompilerParams(
        dimension_semantics=("parallel",)),
)

_tcb = pl.pallas_call(
    _tcb_body,
    grid=(_NBLK,),
    in_specs=[_p_spec, _row_spec(), _dinv_spec, _full((1, _D)),
              _full((_D, _D))],
    out_specs=[_row_spec(), _row_spec()],
    out_shape=[jax.ShapeDtypeStruct((_N_PAD, _D), jnp.float32),
               jax.ShapeDtypeStruct((_N_PAD, _D), jnp.float32)],
    compiler_params=pltpu.CompilerParams(
        dimension_semantics=("parallel",)),
)

_tcc = pl.pallas_call(
    _tcc_body,
    grid=(_NBLK,),
    in_specs=[_p_spec, _row_spec(), _dinv_spec, _full((1, _D)),
              _row_spec(), _row_spec(), _full((3, _D, _D)), _full((1, _D)),
              pl.BlockSpec((1, 1, _BLK), lambda i: (i, 0, 0)),
              _full((_D, _D)), _full((1, _D)), _full((_D, 64)),
              _full((1, 64))],
    out_specs=_full((_G, 64)),
    out_shape=jax.ShapeDtypeStruct((_G, 64), jnp.float32),
    scratch_shapes=[pltpu.VMEM((_G, _D), jnp.float32)],
    compiler_params=pltpu.CompilerParams(
        dimension_semantics=("arbitrary",)),
)


def kernel(x, edge_index, batch, W1, b1, W2, b2, W3, b3, Wjk, bjk,
           Wm1, bm1, Wm2, bm2):
    src = edge_index[0]
    dst = edge_index[1]

    x_pad = jnp.pad(x, ((0, _N_PAD - _N), (0, 0)))
    batch_pad = jnp.pad(batch, (0, _N_PAD - _N),
                        constant_values=jnp.int32(2 ** 30))
    batch3 = batch_pad.reshape(_NBLK, 1, _BLK)
    zeros_tile = jnp.zeros((_RPT, _D), jnp.float32)
    ones_rows = jnp.ones((_K, _DW), jnp.float32)

    deg_parts = _deg_kernel(dst, ones_rows, zeros_tile)
    dinv = _dinv_kernel(deg_parts)
    u1 = _tca(x_pad, dinv, W1)
    p1 = _agg_kernel(u1, src, dst, zeros_tile)
    h1, u2 = _tcb(p1, u1, dinv, b1.reshape(1, _D), W2)
    p2 = _agg_kernel(u2, src, dst, zeros_tile)
    h2, u3 = _tcb(p2, u2, dinv, b2.reshape(1, _D), W3)
    p3 = _agg_kernel(u3, src, dst, zeros_tile)
    out = _tcc(p3, u3, dinv, b3.reshape(1, _D), h1, h2,
               Wjk.reshape(3, _D, _D), bjk.reshape(1, _D), batch3,
               Wm1, bm1.reshape(1, _D), Wm2, bm2.reshape(1, 64))
    return out
